# baseline (speedup 1.0000x reference)
import functools

import jax
import jax.numpy as jnp
from jax import lax
from jax.experimental import pallas as pl
from jax.experimental.pallas import tpu as pltpu
from jax.experimental.pallas import tpu_sc as plsc

N = 10000
E = 320000
T = 12
H = 128
OUT = 64
NS = 16
EB = 128
OCT = 8
TPT = 20
TOT_OCT = NS * TPT
E_PAD = TOT_OCT * OCT * EB
SINK = N
ACC_ROWS = 10240
ZB = 32
ZCH = ACC_ROWS // NS // ZB
ROWS_PT = 624
TAIL = N - NS * ROWS_PT
BN = 400


def _make_segsum(t_steps, d):
    mesh = plsc.VectorSubcoreMesh(core_axis_name="c", subcore_axis_name="s")

    @functools.partial(
        pl.kernel,
        out_type=jax.ShapeDtypeStruct((t_steps * N, d), jnp.float32),
        mesh=mesh,
        scratch_types=[
            pltpu.VMEM((OCT, EB), jnp.int32),
            pltpu.VMEM((OCT, EB), jnp.int32),
            pltpu.VMEM((EB, d), jnp.float32),
            pltpu.VMEM((EB, d), jnp.float32),
            pltpu.VMEM((ZB, d), jnp.float32),
            pltpu.VMEM_SHARED((ACC_ROWS, d), jnp.float32),
            pltpu.SemaphoreType.DMA,
            pltpu.SemaphoreType.DMA,
        ],
    )
    def seg(*refs):
        ys = refs[:t_steps]
        src_hbm, dst_hbm, z_hbm, out = refs[t_steps:t_steps + 4]
        src_c, dst_c, buf0, buf1, zbuf, acc, sem0, sem1 = refs[t_steps + 4:]
        bufs = (buf0, buf1)
        sems = (sem0, sem1)
        c = lax.axis_index("c")
        s = lax.axis_index("s")

        @pl.when(c == 0)
        def _core0_only():
            pltpu.sync_copy(z_hbm, zbuf)
            for t in range(t_steps):
                y = ys[t]
                for z in range(ZCH):
                    pltpu.sync_copy(zbuf,
                                    acc.at[pl.ds((s * ZCH + z) * ZB, ZB)])
                plsc.subcore_barrier()

                def chunk_body(o, carry):
                    row0 = pl.multiple_of((s * TPT + o) * OCT, 8)
                    pltpu.sync_copy(src_hbm.at[pl.ds(row0, OCT)], src_c)
                    pltpu.sync_copy(dst_hbm.at[pl.ds(row0, OCT)], dst_c)
                    cps = [None, None]
                    for b in range(2):
                        cps[b] = pltpu.async_copy(
                            y.at[src_c.at[b]], bufs[b], sems[b])
                    for b in range(OCT):
                        cps[b % 2].wait()
                        pltpu.sync_copy(bufs[b % 2], acc.at[dst_c.at[b]],
                                        add=True)
                        if b + 2 < OCT:
                            cps[b % 2] = pltpu.async_copy(
                                y.at[src_c.at[b + 2]], bufs[b % 2],
                                sems[b % 2])
                    return carry

                lax.fori_loop(0, TPT, chunk_body, 0)
                plsc.subcore_barrier()
                base = t * N
                pltpu.sync_copy(acc.at[pl.ds(s * ROWS_PT, ROWS_PT)],
                                out.at[pl.ds(base + s * ROWS_PT, ROWS_PT)])

                @pl.when(s == NS - 1)
                def _copy_tail():
                    pltpu.sync_copy(acc.at[pl.ds(NS * ROWS_PT, TAIL)],
                                    out.at[pl.ds(base + NS * ROWS_PT, TAIL)])

                plsc.subcore_barrier()

    return seg


def _dinv_body(degp_ref, o_ref):
    o_ref[...] = lax.rsqrt(degp_ref[0:N, 0:1] + 1.0)


def _y0_body(x_ref, w_ref, dinv_ref, o_ref):
    o_ref[0] = dinv_ref[...] * jnp.dot(
        x_ref[0], w_ref[...], preferred_element_type=jnp.float32)


def _mid_body(p0_ref, y0_ref, dinv_ref, b0_ref, w1_ref, o_ref):
    dinv = dinv_ref[...]
    h = jax.nn.relu(dinv * (p0_ref[0] + y0_ref[0]) + b0_ref[...])
    o_ref[0] = dinv * jnp.dot(h, w1_ref[...],
                              preferred_element_type=jnp.float32)


def _lstm_body(p0_ref, y1_ref, dinv_ref, b1_ref, wih_ref, whh_ref,
               bsum_ref, wfc_ref, bfc_ref, o_ref):
    dinv = dinv_ref[...]
    h = jnp.zeros((BN, H), jnp.float32)
    c = jnp.zeros((BN, H), jnp.float32)
    for t in range(T):
        x = jax.nn.relu(dinv * (p0_ref[t] + y1_ref[t]) + b1_ref[...])
        g = (jnp.dot(x, wih_ref[...], preferred_element_type=jnp.float32)
             + jnp.dot(h, whh_ref[...], preferred_element_type=jnp.float32)
             + bsum_ref[...])
        i = jax.nn.sigmoid(g[:, 0:H])
        f = jax.nn.sigmoid(g[:, H:2 * H])
        gg = jnp.tanh(g[:, 2 * H:3 * H])
        o = jax.nn.sigmoid(g[:, 3 * H:4 * H])
        c = f * c + i * gg
        h = o * jnp.tanh(c)
    o_ref[...] = (jnp.dot(h, wfc_ref[...], preferred_element_type=jnp.float32)
                  + bfc_ref[...])


_NB = N // BN


def _tc_dinv(degp):
    return pl.pallas_call(
        _dinv_body,
        out_shape=jax.ShapeDtypeStruct((N, 1), jnp.float32),
    )(degp)


def _tc_y0(xt, W0, dinv):
    return pl.pallas_call(
        _y0_body,
        grid=(T, _NB),
        in_specs=[
            pl.BlockSpec((1, BN, H), lambda t, nb: (t, nb, 0)),
            pl.BlockSpec((H, H), lambda t, nb: (0, 0)),
            pl.BlockSpec((BN, 1), lambda t, nb: (nb, 0)),
        ],
        out_specs=pl.BlockSpec((1, BN, H), lambda t, nb: (t, nb, 0)),
        out_shape=jax.ShapeDtypeStruct((T, N, H), jnp.float32),
    )(xt, W0, dinv)


def _tc_mid(s0, y0, dinv, b0r, W1):
    return pl.pallas_call(
        _mid_body,
        grid=(T, _NB),
        in_specs=[
            pl.BlockSpec((1, BN, H), lambda t, nb: (t, nb, 0)),
            pl.BlockSpec((1, BN, H), lambda t, nb: (t, nb, 0)),
            pl.BlockSpec((BN, 1), lambda t, nb: (nb, 0)),
            pl.BlockSpec((1, H), lambda t, nb: (0, 0)),
            pl.BlockSpec((H, H), lambda t, nb: (0, 0)),
        ],
        out_specs=pl.BlockSpec((1, BN, H), lambda t, nb: (t, nb, 0)),
        out_shape=jax.ShapeDtypeStruct((T, N, H), jnp.float32),
    )(s0, y0, dinv, b0r, W1)


def _tc_lstm(s1, y1, dinv, b1r, wih_t, whh_t, bsum, Wfc, bfcr):
    return pl.pallas_call(
        _lstm_body,
        grid=(_NB,),
        in_specs=[
            pl.BlockSpec((T, BN, H), lambda nb: (0, nb, 0)),
            pl.BlockSpec((T, BN, H), lambda nb: (0, nb, 0)),
            pl.BlockSpec((BN, 1), lambda nb: (nb, 0)),
            pl.BlockSpec((1, H), lambda nb: (0, 0)),
            pl.BlockSpec((H, 4 * H), lambda nb: (0, 0)),
            pl.BlockSpec((H, 4 * H), lambda nb: (0, 0)),
            pl.BlockSpec((1, 4 * H), lambda nb: (0, 0)),
            pl.BlockSpec((H, OUT), lambda nb: (0, 0)),
            pl.BlockSpec((1, OUT), lambda nb: (0, 0)),
        ],
        out_specs=pl.BlockSpec((BN, OUT), lambda nb: (nb, 0)),
        out_shape=jax.ShapeDtypeStruct((N, OUT), jnp.float32),
    )(s1, y1, dinv, b1r, wih_t, whh_t, bsum, Wfc, bfcr)


def kernel(edge_index, x_seq, W0, b0, W1, b1, Wih, Whh, bih, bhh, Wfc, bfc):
    src = edge_index[0].astype(jnp.int32)
    dst = edge_index[1].astype(jnp.int32)
    pad = E_PAD - E
    sink_rows = SINK + (jnp.arange(pad, dtype=jnp.int32) % (ACC_ROWS - N))
    src_p = jnp.concatenate(
        [src, jnp.zeros((pad,), jnp.int32)]).reshape(E_PAD // EB, EB)
    dst_p = jnp.concatenate(
        [dst, sink_rows]).reshape(E_PAD // EB, EB)
    zerosH = jnp.zeros((ZB, H), jnp.float32)
    onesH = jnp.ones((N, H), jnp.float32)

    degp = _make_segsum(1, H)(onesH, src_p, dst_p, zerosH)
    dinv = _tc_dinv(degp)

    xt = jnp.swapaxes(x_seq, 0, 1)
    y0 = _tc_y0(xt, W0, dinv)

    seg = _make_segsum(T, H)
    s0 = seg(*[y0[t] for t in range(T)], src_p, dst_p, zerosH)
    s0 = s0.reshape(T, N, H)
    y1 = _tc_mid(s0, y0, dinv, b0.reshape(1, H), W1)

    s1 = seg(*[y1[t] for t in range(T)], src_p, dst_p, zerosH)
    s1 = s1.reshape(T, N, H)

    out = _tc_lstm(s1, y1, dinv, b1.reshape(1, H), Wih.T, Whh.T,
                   (bih + bhh).reshape(1, 4 * H), Wfc, bfc.reshape(1, OUT))
    return out.reshape(N, 1, OUT)


# speedup vs baseline: 1.4567x; 1.4567x over previous
import functools

import jax
import jax.numpy as jnp
from jax import lax
from jax.experimental import pallas as pl
from jax.experimental.pallas import tpu as pltpu
from jax.experimental.pallas import tpu_sc as plsc

N = 10000
E = 320000
T = 12
H = 128
OUT = 64
NS = 16
EB = 128
OCT = 8
F_OCT = 18
S_OCT = 2
TOT_OCT = NS * (F_OCT + S_OCT)
E_PAD = TOT_OCT * OCT * EB
SINK = N
ACC_ROWS = 10240
ZB = 32
ZCH = ACC_ROWS // NS // ZB
ROWS_PT = 624
TAIL = N - NS * ROWS_PT
BN = 400


def _make_segsum(t_steps, d):
    mesh = plsc.VectorSubcoreMesh(core_axis_name="c", subcore_axis_name="s")

    @functools.partial(
        pl.kernel,
        out_type=jax.ShapeDtypeStruct((t_steps * 2 * N, d), jnp.float32),
        mesh=mesh,
        scratch_types=[
            pltpu.VMEM((OCT, EB), jnp.int32),
            pltpu.VMEM((OCT, EB), jnp.int32),
            pltpu.VMEM((EB, d), jnp.float32),
            pltpu.VMEM((EB, d), jnp.float32),
            pltpu.VMEM((ZB, d), jnp.float32),
            pltpu.VMEM_SHARED((ACC_ROWS, d), jnp.float32),
            pltpu.SemaphoreType.DMA,
            pltpu.SemaphoreType.DMA,
        ],
    )
    def seg(*refs):
        ys = refs[:t_steps]
        src_hbm, dst_hbm, z_hbm, out = refs[t_steps:t_steps + 4]
        src_c, dst_c, buf0, buf1, zbuf, acc, sem0, sem1 = refs[t_steps + 4:]
        bufs = (buf0, buf1)
        sems = (sem0, sem1)
        c = lax.axis_index("c")
        s = lax.axis_index("s")
        n_oct = jnp.where(c == 0, F_OCT, S_OCT)
        oct_base = jnp.where(c == 0, s * F_OCT, NS * F_OCT + s * S_OCT)
        pltpu.sync_copy(z_hbm, zbuf)
        for t in range(t_steps):
            y = ys[t]
            for z in range(ZCH):
                pltpu.sync_copy(zbuf, acc.at[pl.ds((s * ZCH + z) * ZB, ZB)])
            plsc.subcore_barrier()

            def chunk_body(o, carry):
                row0 = pl.multiple_of((oct_base + o) * OCT, 8)
                pltpu.sync_copy(src_hbm.at[pl.ds(row0, OCT)], src_c)
                pltpu.sync_copy(dst_hbm.at[pl.ds(row0, OCT)], dst_c)
                cps = [None, None]
                for b in range(2):
                    cps[b] = pltpu.async_copy(
                        y.at[src_c.at[b]], bufs[b], sems[b])
                for b in range(OCT):
                    cps[b % 2].wait()
                    pltpu.sync_copy(bufs[b % 2], acc.at[dst_c.at[b]],
                                    add=True)
                    if b + 2 < OCT:
                        cps[b % 2] = pltpu.async_copy(
                            y.at[src_c.at[b + 2]], bufs[b % 2], sems[b % 2])
                return carry

            lax.fori_loop(0, n_oct, chunk_body, 0)
            plsc.subcore_barrier()
            base = t * (2 * N) + c * N
            pltpu.sync_copy(acc.at[pl.ds(s * ROWS_PT, ROWS_PT)],
                            out.at[pl.ds(base + s * ROWS_PT, ROWS_PT)])

            @pl.when(s == NS - 1)
            def _copy_tail():
                pltpu.sync_copy(acc.at[pl.ds(NS * ROWS_PT, TAIL)],
                                out.at[pl.ds(base + NS * ROWS_PT, TAIL)])

            plsc.subcore_barrier()

    return seg


def _dinv_body(degp_ref, o_ref):
    o_ref[...] = lax.rsqrt(
        degp_ref[0:N, 0:1] + degp_ref[N:2 * N, 0:1] + 1.0)


def _y0_body(x_ref, w_ref, dinv_ref, o_ref):
    o_ref[0] = dinv_ref[...] * jnp.dot(
        x_ref[0], w_ref[...], preferred_element_type=jnp.float32)


def _mid_body(p0_ref, p1_ref, y0_ref, dinv_ref, b0_ref, w1_ref, o_ref):
    dinv = dinv_ref[...]
    h = jax.nn.relu(dinv * (p0_ref[0] + p1_ref[0] + y0_ref[0]) + b0_ref[...])
    o_ref[0] = dinv * jnp.dot(h, w1_ref[...],
                              preferred_element_type=jnp.float32)


def _lstm_body(p0_ref, p1_ref, y1_ref, dinv_ref, b1_ref, wih_ref, whh_ref,
               bsum_ref, wfc_ref, bfc_ref, o_ref):
    dinv = dinv_ref[...]
    h = jnp.zeros((BN, H), jnp.float32)
    c = jnp.zeros((BN, H), jnp.float32)
    for t in range(T):
        x = jax.nn.relu(dinv * (p0_ref[t] + p1_ref[t] + y1_ref[t])
                        + b1_ref[...])
        g = (jnp.dot(x, wih_ref[...], preferred_element_type=jnp.float32)
             + jnp.dot(h, whh_ref[...], preferred_element_type=jnp.float32)
             + bsum_ref[...])
        i = jax.nn.sigmoid(g[:, 0:H])
        f = jax.nn.sigmoid(g[:, H:2 * H])
        gg = jnp.tanh(g[:, 2 * H:3 * H])
        o = jax.nn.sigmoid(g[:, 3 * H:4 * H])
        c = f * c + i * gg
        h = o * jnp.tanh(c)
    o_ref[...] = (jnp.dot(h, wfc_ref[...], preferred_element_type=jnp.float32)
                  + bfc_ref[...])


_NB = N // BN


def _tc_dinv(degp):
    return pl.pallas_call(
        _dinv_body,
        out_shape=jax.ShapeDtypeStruct((N, 1), jnp.float32),
    )(degp)


def _tc_y0(xt, W0, dinv):
    return pl.pallas_call(
        _y0_body,
        grid=(T, _NB),
        in_specs=[
            pl.BlockSpec((1, BN, H), lambda t, nb: (t, nb, 0)),
            pl.BlockSpec((H, H), lambda t, nb: (0, 0)),
            pl.BlockSpec((BN, 1), lambda t, nb: (nb, 0)),
        ],
        out_specs=pl.BlockSpec((1, BN, H), lambda t, nb: (t, nb, 0)),
        out_shape=jax.ShapeDtypeStruct((T, N, H), jnp.float32),
    )(xt, W0, dinv)


def _tc_mid(s0, y0, dinv, b0r, W1):
    return pl.pallas_call(
        _mid_body,
        grid=(T, _NB),
        in_specs=[
            pl.BlockSpec((1, BN, H), lambda t, nb: (t, nb, 0)),
            pl.BlockSpec((1, BN, H), lambda t, nb: (t, _NB + nb, 0)),
            pl.BlockSpec((1, BN, H), lambda t, nb: (t, nb, 0)),
            pl.BlockSpec((BN, 1), lambda t, nb: (nb, 0)),
            pl.BlockSpec((1, H), lambda t, nb: (0, 0)),
            pl.BlockSpec((H, H), lambda t, nb: (0, 0)),
        ],
        out_specs=pl.BlockSpec((1, BN, H), lambda t, nb: (t, nb, 0)),
        out_shape=jax.ShapeDtypeStruct((T, N, H), jnp.float32),
    )(s0, s0, y0, dinv, b0r, W1)


def _tc_lstm(s1, y1, dinv, b1r, wih_t, whh_t, bsum, Wfc, bfcr):
    return pl.pallas_call(
        _lstm_body,
        grid=(_NB,),
        in_specs=[
            pl.BlockSpec((T, BN, H), lambda nb: (0, nb, 0)),
            pl.BlockSpec((T, BN, H), lambda nb: (0, _NB + nb, 0)),
            pl.BlockSpec((T, BN, H), lambda nb: (0, nb, 0)),
            pl.BlockSpec((BN, 1), lambda nb: (nb, 0)),
            pl.BlockSpec((1, H), lambda nb: (0, 0)),
            pl.BlockSpec((H, 4 * H), lambda nb: (0, 0)),
            pl.BlockSpec((H, 4 * H), lambda nb: (0, 0)),
            pl.BlockSpec((1, 4 * H), lambda nb: (0, 0)),
            pl.BlockSpec((H, OUT), lambda nb: (0, 0)),
            pl.BlockSpec((1, OUT), lambda nb: (0, 0)),
        ],
        out_specs=pl.BlockSpec((BN, OUT), lambda nb: (nb, 0)),
        out_shape=jax.ShapeDtypeStruct((N, OUT), jnp.float32),
    )(s1, s1, y1, dinv, b1r, wih_t, whh_t, bsum, Wfc, bfcr)


def kernel(edge_index, x_seq, W0, b0, W1, b1, Wih, Whh, bih, bhh, Wfc, bfc):
    src = edge_index[0].astype(jnp.int32)
    dst = edge_index[1].astype(jnp.int32)
    pad = E_PAD - E
    sink_rows = SINK + (jnp.arange(pad, dtype=jnp.int32) % (ACC_ROWS - N))
    src_p = jnp.concatenate(
        [src, jnp.zeros((pad,), jnp.int32)]).reshape(E_PAD // EB, EB)
    dst_p = jnp.concatenate(
        [dst, sink_rows]).reshape(E_PAD // EB, EB)
    zerosH = jnp.zeros((ZB, H), jnp.float32)
    onesH = jnp.ones((N, H), jnp.float32)

    degp = _make_segsum(1, H)(onesH, src_p, dst_p, zerosH)
    dinv = _tc_dinv(degp)

    xt = jnp.swapaxes(x_seq, 0, 1)
    y0 = _tc_y0(xt, W0, dinv)

    seg = _make_segsum(T, H)
    s0 = seg(*[y0[t] for t in range(T)], src_p, dst_p, zerosH)
    s0 = s0.reshape(T, 2 * N, H)
    y1 = _tc_mid(s0, y0, dinv, b0.reshape(1, H), W1)

    s1 = seg(*[y1[t] for t in range(T)], src_p, dst_p, zerosH)
    s1 = s1.reshape(T, 2 * N, H)

    out = _tc_lstm(s1, y1, dinv, b1.reshape(1, H), Wih.T, Whh.T,
                   (bih + bhh).reshape(1, 4 * H), Wfc, bfc.reshape(1, OUT))
    return out.reshape(N, 1, OUT)


# speedup vs baseline: 1.4598x; 1.0021x over previous
import functools

import jax
import jax.numpy as jnp
from jax import lax
from jax.experimental import pallas as pl
from jax.experimental.pallas import tpu as pltpu
from jax.experimental.pallas import tpu_sc as plsc

N = 10000
E = 320000
T = 12
H = 128
OUT = 64
NS = 16
EB = 128
OCT = 8
F_OCT = 18
S_OCT = 2
TOT_OCT = NS * (F_OCT + S_OCT)
E_PAD = TOT_OCT * OCT * EB
SINK = N
ACC_ROWS = 10240
ZB = 32
ZCH = ACC_ROWS // NS // ZB
ROWS_PT = 624
CPR = 104
TAIL = N - NS * ROWS_PT
BN = 400


def _make_segsum(t_steps, d):
    mesh = plsc.VectorSubcoreMesh(core_axis_name="c", subcore_axis_name="s")

    @functools.partial(
        pl.kernel,
        out_type=jax.ShapeDtypeStruct((t_steps * 2 * N, d), jnp.float32),
        mesh=mesh,
        scratch_types=[
            pltpu.VMEM((OCT, EB), jnp.int32),
            pltpu.VMEM((OCT, EB), jnp.int32),
            pltpu.VMEM((EB, d), jnp.float32),
            pltpu.VMEM((EB, d), jnp.float32),
            pltpu.VMEM((ZB, d), jnp.float32),
            pltpu.VMEM_SHARED((ACC_ROWS, d), jnp.float32),
            pltpu.SemaphoreType.DMA,
            pltpu.SemaphoreType.DMA,
        ],
    )
    def seg(*refs):
        ys = refs[:t_steps]
        src_hbm, dst_hbm, z_hbm, out = refs[t_steps:t_steps + 4]
        src_c, dst_c, buf0, buf1, zbuf, acc, sem0, sem1 = refs[t_steps + 4:]
        bufs = (buf0, buf1)
        sems = (sem0, sem1)
        c = lax.axis_index("c")
        s = lax.axis_index("s")
        n_oct = jnp.where(c == 0, F_OCT, S_OCT)
        oct_base = jnp.where(c == 0, s * F_OCT, NS * F_OCT + s * S_OCT)
        pltpu.sync_copy(z_hbm, zbuf)
        for t in range(t_steps):
            y = ys[t]
            for z in range(ZCH):
                pltpu.sync_copy(zbuf, acc.at[pl.ds((s * ZCH + z) * ZB, ZB)])
            plsc.subcore_barrier()

            def chunk_body(o, carry):
                row0 = pl.multiple_of((oct_base + o) * OCT, 8)
                pltpu.sync_copy(src_hbm.at[pl.ds(row0, OCT)], src_c)
                pltpu.sync_copy(dst_hbm.at[pl.ds(row0, OCT)], dst_c)
                cps = [None, None]
                for b in range(2):
                    cps[b] = pltpu.async_copy(
                        y.at[src_c.at[b]], bufs[b], sems[b])
                for b in range(OCT):
                    cps[b % 2].wait()
                    pltpu.sync_copy(bufs[b % 2], acc.at[dst_c.at[b]],
                                    add=True)
                    if b + 2 < OCT:
                        cps[b % 2] = pltpu.async_copy(
                            y.at[src_c.at[b + 2]], bufs[b % 2], sems[b % 2])
                return carry

            lax.fori_loop(0, n_oct, chunk_body, 0)
            plsc.subcore_barrier()
            base = t * (2 * N) + c * N
            cps = [None, None]
            for k in range(ROWS_PT // CPR):
                bb = bufs[k % 2]
                if cps[k % 2] is not None:
                    cps[k % 2].wait()
                off = s * ROWS_PT + k * CPR
                pltpu.sync_copy(acc.at[pl.ds(off, CPR)],
                                bb.at[pl.ds(0, CPR)])
                cps[k % 2] = pltpu.async_copy(
                    bb.at[pl.ds(0, CPR)], out.at[pl.ds(base + off, CPR)],
                    sems[k % 2])
            for cp in cps:
                cp.wait()

            @pl.when(s == NS - 1)
            def _copy_tail():
                pltpu.sync_copy(acc.at[pl.ds(NS * ROWS_PT, TAIL)],
                                bufs[0].at[pl.ds(0, TAIL)])
                pltpu.sync_copy(bufs[0].at[pl.ds(0, TAIL)],
                                out.at[pl.ds(base + NS * ROWS_PT, TAIL)])

            plsc.subcore_barrier()

    return seg


def _dinv_body(degp_ref, o_ref):
    o_ref[...] = lax.rsqrt(
        degp_ref[0:N, 0:1] + degp_ref[N:2 * N, 0:1] + 1.0)


def _y0_body(x_ref, w_ref, dinv_ref, o_ref):
    o_ref[0] = dinv_ref[...] * jnp.dot(
        x_ref[0], w_ref[...], preferred_element_type=jnp.float32)


def _mid_body(p0_ref, p1_ref, y0_ref, dinv_ref, b0_ref, w1_ref, o_ref):
    dinv = dinv_ref[...]
    h = jax.nn.relu(dinv * (p0_ref[0] + p1_ref[0] + y0_ref[0]) + b0_ref[...])
    o_ref[0] = dinv * jnp.dot(h, w1_ref[...],
                              preferred_element_type=jnp.float32)


def _lstm_body(p0_ref, p1_ref, y1_ref, dinv_ref, b1_ref, wih_ref, whh_ref,
               bsum_ref, wfc_ref, bfc_ref, o_ref):
    dinv = dinv_ref[...]
    h = jnp.zeros((BN, H), jnp.float32)
    c = jnp.zeros((BN, H), jnp.float32)
    for t in range(T):
        x = jax.nn.relu(dinv * (p0_ref[t] + p1_ref[t] + y1_ref[t])
                        + b1_ref[...])
        g = (jnp.dot(x, wih_ref[...], preferred_element_type=jnp.float32)
             + jnp.dot(h, whh_ref[...], preferred_element_type=jnp.float32)
             + bsum_ref[...])
        i = jax.nn.sigmoid(g[:, 0:H])
        f = jax.nn.sigmoid(g[:, H:2 * H])
        gg = jnp.tanh(g[:, 2 * H:3 * H])
        o = jax.nn.sigmoid(g[:, 3 * H:4 * H])
        c = f * c + i * gg
        h = o * jnp.tanh(c)
    o_ref[...] = (jnp.dot(h, wfc_ref[...], preferred_element_type=jnp.float32)
                  + bfc_ref[...])


_NB = N // BN


def _tc_dinv(degp):
    return pl.pallas_call(
        _dinv_body,
        out_shape=jax.ShapeDtypeStruct((N, 1), jnp.float32),
    )(degp)


def _tc_y0(xt, W0, dinv):
    return pl.pallas_call(
        _y0_body,
        grid=(T, _NB),
        in_specs=[
            pl.BlockSpec((1, BN, H), lambda t, nb: (t, nb, 0)),
            pl.BlockSpec((H, H), lambda t, nb: (0, 0)),
            pl.BlockSpec((BN, 1), lambda t, nb: (nb, 0)),
        ],
        out_specs=pl.BlockSpec((1, BN, H), lambda t, nb: (t, nb, 0)),
        out_shape=jax.ShapeDtypeStruct((T, N, H), jnp.float32),
    )(xt, W0, dinv)


def _tc_mid(s0, y0, dinv, b0r, W1):
    return pl.pallas_call(
        _mid_body,
        grid=(T, _NB),
        in_specs=[
            pl.BlockSpec((1, BN, H), lambda t, nb: (t, nb, 0)),
            pl.BlockSpec((1, BN, H), lambda t, nb: (t, _NB + nb, 0)),
            pl.BlockSpec((1, BN, H), lambda t, nb: (t, nb, 0)),
            pl.BlockSpec((BN, 1), lambda t, nb: (nb, 0)),
            pl.BlockSpec((1, H), lambda t, nb: (0, 0)),
            pl.BlockSpec((H, H), lambda t, nb: (0, 0)),
        ],
        out_specs=pl.BlockSpec((1, BN, H), lambda t, nb: (t, nb, 0)),
        out_shape=jax.ShapeDtypeStruct((T, N, H), jnp.float32),
    )(s0, s0, y0, dinv, b0r, W1)


def _tc_lstm(s1, y1, dinv, b1r, wih_t, whh_t, bsum, Wfc, bfcr):
    return pl.pallas_call(
        _lstm_body,
        grid=(_NB,),
        in_specs=[
            pl.BlockSpec((T, BN, H), lambda nb: (0, nb, 0)),
            pl.BlockSpec((T, BN, H), lambda nb: (0, _NB + nb, 0)),
            pl.BlockSpec((T, BN, H), lambda nb: (0, nb, 0)),
            pl.BlockSpec((BN, 1), lambda nb: (nb, 0)),
            pl.BlockSpec((1, H), lambda nb: (0, 0)),
            pl.BlockSpec((H, 4 * H), lambda nb: (0, 0)),
            pl.BlockSpec((H, 4 * H), lambda nb: (0, 0)),
            pl.BlockSpec((1, 4 * H), lambda nb: (0, 0)),
            pl.BlockSpec((H, OUT), lambda nb: (0, 0)),
            pl.BlockSpec((1, OUT), lambda nb: (0, 0)),
        ],
        out_specs=pl.BlockSpec((BN, OUT), lambda nb: (nb, 0)),
        out_shape=jax.ShapeDtypeStruct((N, OUT), jnp.float32),
    )(s1, s1, y1, dinv, b1r, wih_t, whh_t, bsum, Wfc, bfcr)


def kernel(edge_index, x_seq, W0, b0, W1, b1, Wih, Whh, bih, bhh, Wfc, bfc):
    src = edge_index[0].astype(jnp.int32)
    dst = edge_index[1].astype(jnp.int32)
    pad = E_PAD - E
    sink_rows = SINK + (jnp.arange(pad, dtype=jnp.int32) % (ACC_ROWS - N))
    src_p = jnp.concatenate(
        [src, jnp.zeros((pad,), jnp.int32)]).reshape(E_PAD // EB, EB)
    dst_p = jnp.concatenate(
        [dst, sink_rows]).reshape(E_PAD // EB, EB)
    zerosH = jnp.zeros((ZB, H), jnp.float32)
    onesH = jnp.ones((N, H), jnp.float32)

    degp = _make_segsum(1, H)(onesH, src_p, dst_p, zerosH)
    dinv = _tc_dinv(degp)

    xt = jnp.swapaxes(x_seq, 0, 1)
    y0 = _tc_y0(xt, W0, dinv)

    seg = _make_segsum(T, H)
    s0 = seg(*[y0[t] for t in range(T)], src_p, dst_p, zerosH)
    s0 = s0.reshape(T, 2 * N, H)
    y1 = _tc_mid(s0, y0, dinv, b0.reshape(1, H), W1)

    s1 = seg(*[y1[t] for t in range(T)], src_p, dst_p, zerosH)
    s1 = s1.reshape(T, 2 * N, H)

    out = _tc_lstm(s1, y1, dinv, b1.reshape(1, H), Wih.T, Whh.T,
                   (bih + bhh).reshape(1, 4 * H), Wfc, bfc.reshape(1, OUT))
    return out.reshape(N, 1, OUT)
